# Initial kernel scaffold; baseline (speedup 1.0000x reference)
import jax
import jax.numpy as jnp
from jax.experimental import pallas as pl


def kernel(x, index):
    raise NotImplementedError("write your pallas kernel here")


# speedup vs baseline: 3.6252x; 3.6252x over previous
import functools

import jax
import jax.numpy as jnp
from jax import lax
from jax.experimental import pallas as pl
from jax.experimental.pallas import tpu as pltpu
from jax.experimental.pallas import tpu_sc as plsc

N_EDGES = 320000
D = 128
N_SEG = 10000
N_SEG_PAD = 10240
CHUNK = 80
NC = 2
NS = 16
NW = NC * NS
EDGES_PER_W = N_EDGES // NW
ITERS = EDGES_PER_W // CHUNK
ROWS_PER_SUB = N_SEG_PAD // NS


def _sc_partial_sums(x, idx, zeros):
    mesh = plsc.VectorSubcoreMesh(core_axis_name="c", subcore_axis_name="s")

    @functools.partial(
        pl.kernel,
        mesh=mesh,
        out_type=jax.ShapeDtypeStruct((NC, N_SEG_PAD, D), jnp.float32),
        scratch_types=[
            pltpu.VMEM((CHUNK, D), jnp.float32),
            pltpu.VMEM((CHUNK,), jnp.int32),
            pltpu.VMEM_SHARED((N_SEG_PAD, D), jnp.float32),
        ],
    )
    def k(x_hbm, idx_hbm, z_hbm, out_hbm, xbuf, ibuf, acc):
        cid = lax.axis_index("c")
        sid = lax.axis_index("s")
        wid = sid * NC + cid
        pltpu.sync_copy(
            z_hbm.at[pl.ds(sid * ROWS_PER_SUB, ROWS_PER_SUB)],
            acc.at[pl.ds(sid * ROWS_PER_SUB, ROWS_PER_SUB)],
        )
        plsc.subcore_barrier()

        base = wid * EDGES_PER_W

        def body(i, carry):
            off = base + i * CHUNK
            pltpu.sync_copy(x_hbm.at[pl.ds(off, CHUNK)], xbuf)
            pltpu.sync_copy(idx_hbm.at[pl.ds(off, CHUNK)], ibuf)
            pltpu.sync_copy(xbuf, acc.at[ibuf], add=True)
            return carry

        lax.fori_loop(0, ITERS, body, 0)
        plsc.subcore_barrier()
        pltpu.sync_copy(
            acc.at[pl.ds(sid * ROWS_PER_SUB, ROWS_PER_SUB)],
            out_hbm.at[cid, pl.ds(sid * ROWS_PER_SUB, ROWS_PER_SUB)],
        )

    return k(x, idx, zeros)


def _tc_add(a, b):
    def body(a_ref, b_ref, o_ref):
        o_ref[...] = a_ref[...] + b_ref[...]

    return pl.pallas_call(
        body,
        grid=(10,),
        in_specs=[
            pl.BlockSpec((1000, D), lambda i: (i, 0)),
            pl.BlockSpec((1000, D), lambda i: (i, 0)),
        ],
        out_specs=pl.BlockSpec((1000, D), lambda i: (i, 0)),
        out_shape=jax.ShapeDtypeStruct((N_SEG, D), jnp.float32),
    )(a, b)


def kernel(x, index):
    idx = index.astype(jnp.int32)
    zeros = jnp.zeros((N_SEG_PAD, D), jnp.float32)
    parts = _sc_partial_sums(x, idx, zeros)
    return _tc_add(parts[0], parts[1])


# speedup vs baseline: 6.7279x; 1.8559x over previous
import functools

import jax
import jax.numpy as jnp
from jax import lax
from jax.experimental import pallas as pl
from jax.experimental.pallas import tpu as pltpu
from jax.experimental.pallas import tpu_sc as plsc

N_EDGES = 320000
D = 128
N_SEG = 10000
N_SEG_PAD = 10240
CHUNK = 80
NC = 2
NS = 16
NW = NC * NS
EDGES_PER_W = N_EDGES // NW
ITERS = EDGES_PER_W // CHUNK
ROWS_PER_SUB = N_SEG_PAD // NS


def _sc_partial_sums(x, idx, zeros):
    mesh = plsc.VectorSubcoreMesh(core_axis_name="c", subcore_axis_name="s")

    @functools.partial(
        pl.kernel,
        mesh=mesh,
        out_type=jax.ShapeDtypeStruct((NC, N_SEG_PAD, D), jnp.float32),
        scratch_types=[
            pltpu.VMEM((CHUNK, D), jnp.float32),
            pltpu.VMEM((CHUNK, D), jnp.float32),
            pltpu.VMEM((ITERS, CHUNK), jnp.int32),
            pltpu.VMEM_SHARED((N_SEG_PAD, D), jnp.float32),
            pltpu.SemaphoreType.DMA,
            pltpu.SemaphoreType.DMA,
        ],
    )
    def k(x_hbm, idx_hbm, z_hbm, out_hbm, xbuf0, xbuf1, ibuf, acc, sem0, sem1):
        cid = lax.axis_index("c")
        sid = lax.axis_index("s")
        wid = sid * NC + cid
        base = wid * EDGES_PER_W
        pltpu.sync_copy(idx_hbm.at[wid], ibuf)
        pltpu.sync_copy(
            z_hbm.at[pl.ds(sid * ROWS_PER_SUB, ROWS_PER_SUB)],
            acc.at[pl.ds(sid * ROWS_PER_SUB, ROWS_PER_SUB)],
        )
        plsc.subcore_barrier()

        def load(i, buf, sem):
            return pltpu.async_copy(x_hbm.at[pl.ds(base + i * CHUNK, CHUNK)], buf, sem)

        load(0, xbuf0, sem0)

        def body(g, carry):
            c = 2 * g
            load(c + 1, xbuf1, sem1)
            pltpu.make_async_copy(x_hbm.at[pl.ds(base + c * CHUNK, CHUNK)], xbuf0, sem0).wait()
            pltpu.sync_copy(xbuf0, acc.at[ibuf.at[c]], add=True)
            load(c + 2, xbuf0, sem0)
            pltpu.make_async_copy(x_hbm.at[pl.ds(base + (c + 1) * CHUNK, CHUNK)], xbuf1, sem1).wait()
            pltpu.sync_copy(xbuf1, acc.at[ibuf.at[c + 1]], add=True)
            return carry

        lax.fori_loop(0, (ITERS - 1) // 2, body, 0)
        pltpu.make_async_copy(
            x_hbm.at[pl.ds(base + (ITERS - 1) * CHUNK, CHUNK)], xbuf0, sem0
        ).wait()
        pltpu.sync_copy(xbuf0, acc.at[ibuf.at[ITERS - 1]], add=True)
        plsc.subcore_barrier()
        pltpu.sync_copy(
            acc.at[pl.ds(sid * ROWS_PER_SUB, ROWS_PER_SUB)],
            out_hbm.at[cid, pl.ds(sid * ROWS_PER_SUB, ROWS_PER_SUB)],
        )

    return k(x, idx, zeros)


def _tc_add(a, b):
    def body(a_ref, b_ref, o_ref):
        o_ref[...] = a_ref[...] + b_ref[...]

    return pl.pallas_call(
        body,
        grid=(10,),
        in_specs=[
            pl.BlockSpec((1000, D), lambda i: (i, 0)),
            pl.BlockSpec((1000, D), lambda i: (i, 0)),
        ],
        out_specs=pl.BlockSpec((1000, D), lambda i: (i, 0)),
        out_shape=jax.ShapeDtypeStruct((N_SEG, D), jnp.float32),
    )(a, b)


def kernel(x, index):
    idx = index.astype(jnp.int32).reshape(NW, ITERS, CHUNK)
    zeros = jnp.zeros((N_SEG_PAD, D), jnp.float32)
    parts = _sc_partial_sums(x, idx, zeros)
    return _tc_add(parts[0], parts[1])
